# Initial kernel scaffold; baseline (speedup 1.0000x reference)
import jax
import jax.numpy as jnp
from jax.experimental import pallas as pl


def kernel(x_num, x_cat, emb_tables, W1, b1, W2, b2, W3, b3):
    raise NotImplementedError("write your pallas kernel here")


# speedup vs baseline: 7.5399x; 7.5399x over previous
import functools

import jax
import jax.numpy as jnp
from jax import lax
from jax.experimental import pallas as pl
from jax.experimental.pallas import tpu as pltpu
from jax.experimental.pallas import tpu_sc as plsc

B = 16384
F = 26
V = 100000
D = 16
NUM = 16

NC = 2
NS = 16
NW = NC * NS
BF = B * F
PW = BF // NW
NCH = 8
CH = PW // NCH


def _gather_body(tbl_hbm, idx_hbm, out_hbm, idx_v, buf0, buf1,
                 gs0, gs1, os0, os1):
    wid = lax.axis_index("s") * NC + lax.axis_index("c")
    base = wid * PW
    pltpu.sync_copy(idx_hbm.at[pl.ds(base, PW)], idx_v)
    bufs = (buf0, buf1)
    gsem = (gs0, gs1)
    osem = (os0, os1)
    gcp = [None, None]
    ocp = [None, None]
    for c in range(NCH):
        b = c & 1
        if ocp[b] is not None:
            ocp[b].wait()
        gcp[b] = pltpu.async_copy(
            tbl_hbm.at[idx_v.at[pl.ds(c * CH, CH)]], bufs[b], gsem[b])
        if c > 0:
            pb = 1 - b
            gcp[pb].wait()
            ocp[pb] = pltpu.async_copy(
                bufs[pb], out_hbm.at[pl.ds(base + (c - 1) * CH, CH)], osem[pb])
    lb = (NCH - 1) & 1
    gcp[lb].wait()
    ocp[lb] = pltpu.async_copy(
        bufs[lb], out_hbm.at[pl.ds(base + (NCH - 1) * CH, CH)], osem[lb])
    ocp[1 - lb].wait()
    ocp[lb].wait()


def _sc_gather(tbl2d, idx_flat):
    mesh = plsc.VectorSubcoreMesh(core_axis_name="c", subcore_axis_name="s")
    f = pl.kernel(
        _gather_body,
        out_type=jax.ShapeDtypeStruct((BF, D), jnp.float32),
        mesh=mesh,
        compiler_params=pltpu.CompilerParams(use_tc_tiling_on_sc=False),
        scratch_types=[
            pltpu.VMEM((PW,), jnp.int32),
            pltpu.VMEM((CH, D), jnp.float32),
            pltpu.VMEM((CH, D), jnp.float32),
            pltpu.SemaphoreType.DMA,
            pltpu.SemaphoreType.DMA,
            pltpu.SemaphoreType.DMA,
            pltpu.SemaphoreType.DMA,
        ],
    )
    return f(tbl2d, idx_flat)


def _mlp_body(xn_ref, e_ref, w1a_ref, w1b_ref, b1_ref, w2_ref, b2_ref,
              w3_ref, b3_ref, o_ref):
    hp = jax.lax.Precision.HIGHEST
    h = jnp.dot(xn_ref[...], w1a_ref[...],
                preferred_element_type=jnp.float32, precision=hp)
    h = h + jnp.dot(e_ref[...], w1b_ref[...],
                    preferred_element_type=jnp.float32, precision=hp)
    h = jnp.maximum(h + b1_ref[...], 0.0)
    h = jnp.dot(h, w2_ref[...], preferred_element_type=jnp.float32,
                precision=hp)
    h = jnp.maximum(h + b2_ref[...], 0.0)
    o_ref[...] = jnp.dot(h, w3_ref[...], preferred_element_type=jnp.float32,
                         precision=hp) + b3_ref[...]


def _tc_mlp(x_num, embs, W1, b1, W2, b2, W3, b3):
    BK = 2048
    in_dim = NUM + F * D
    w1a = W1[:NUM]
    w1b = W1[NUM:]
    grid = (B // BK,)
    return pl.pallas_call(
        _mlp_body,
        grid=grid,
        in_specs=[
            pl.BlockSpec((BK, NUM), lambda i: (i, 0)),
            pl.BlockSpec((BK, F * D), lambda i: (i, 0)),
            pl.BlockSpec((NUM, 64), lambda i: (0, 0)),
            pl.BlockSpec((in_dim - NUM, 64), lambda i: (0, 0)),
            pl.BlockSpec((1, 64), lambda i: (0, 0)),
            pl.BlockSpec((64, 32), lambda i: (0, 0)),
            pl.BlockSpec((1, 32), lambda i: (0, 0)),
            pl.BlockSpec((32, 1), lambda i: (0, 0)),
            pl.BlockSpec((1, 1), lambda i: (0, 0)),
        ],
        out_specs=pl.BlockSpec((BK, 1), lambda i: (i, 0)),
        out_shape=jax.ShapeDtypeStruct((B, 1), jnp.float32),
    )(x_num, embs, w1a, w1b, b1.reshape(1, 64), W2, b2.reshape(1, 32),
      W3, b3.reshape(1, 1))


def kernel(x_num, x_cat, emb_tables, W1, b1, W2, b2, W3, b3):
    tbl2d = emb_tables.reshape(F * V, D)
    idx_flat = (x_cat.astype(jnp.int32)
                + (jnp.arange(F, dtype=jnp.int32) * V)[None, :]).reshape(-1)
    rows = _sc_gather(tbl2d, idx_flat)
    embs = rows.reshape(B, F * D)
    return _tc_mlp(x_num, embs, W1, b1, W2, b2, W3, b3)


# speedup vs baseline: 38.9569x; 5.1667x over previous
import functools

import jax
import jax.numpy as jnp
from jax import lax
from jax.experimental import pallas as pl
from jax.experimental.pallas import tpu as pltpu
from jax.experimental.pallas import tpu_sc as plsc

B = 16384
F = 26
V = 100000
D = 16
NUM = 16

NC = 2
NS = 16
NW = NC * NS
R = F * D
PW = R // NW
HB = B // 2


def _gather_body(tbl_hbm, xcat_hbm, out_hbm, row_v, idx_v, res_v, sem):
    wid = lax.axis_index("s") * NC + lax.axis_index("c")
    for j in range(PW):
        p = wid * PW + j
        f = p >> 4
        pltpu.async_copy(tbl_hbm.at[p], row_v, sem).wait()
        for h in range(2):
            pltpu.async_copy(xcat_hbm.at[f, pl.ds(h * HB, HB)], idx_v,
                             sem).wait()

            def gather_step(i, _):
                vec = idx_v[pl.ds(i * 16, 16)]
                res_v[pl.ds(i * 16, 16)] = plsc.load_gather(row_v, [vec])
                return 0

            lax.fori_loop(0, HB // 16, gather_step, 0)
            pltpu.async_copy(res_v, out_hbm.at[p, pl.ds(h * HB, HB)],
                             sem).wait()


def _sc_gather(tblT, xcatT):
    mesh = plsc.VectorSubcoreMesh(core_axis_name="c", subcore_axis_name="s")
    f = pl.kernel(
        _gather_body,
        out_type=jax.ShapeDtypeStruct((R, B), jnp.float32),
        mesh=mesh,
        compiler_params=pltpu.CompilerParams(use_tc_tiling_on_sc=True,
                                             needs_layout_passes=False),
        scratch_types=[
            pltpu.VMEM((V,), jnp.float32),
            pltpu.VMEM((HB,), jnp.int32),
            pltpu.VMEM((HB,), jnp.float32),
            pltpu.SemaphoreType.DMA,
        ],
    )
    return f(tblT, xcatT)


def _mlp_body(xnt_ref, et_ref, w1t_ref, b1_ref, w2t_ref, b2_ref, w3t_ref,
              b3_ref, o_ref):
    hp = jax.lax.Precision.HIGHEST
    h = jnp.dot(w1t_ref[:, :NUM], xnt_ref[...],
                preferred_element_type=jnp.float32, precision=hp)
    h = h + jnp.dot(w1t_ref[:, NUM:], et_ref[...],
                    preferred_element_type=jnp.float32, precision=hp)
    h = jnp.maximum(h + b1_ref[...], 0.0)
    h = jnp.dot(w2t_ref[...], h, preferred_element_type=jnp.float32,
                precision=hp)
    h = jnp.maximum(h + b2_ref[...], 0.0)
    o_ref[...] = jnp.dot(w3t_ref[...], h, preferred_element_type=jnp.float32,
                         precision=hp) + b3_ref[...]


def _tc_mlp(xnT, embsT, W1T, b1, W2T, b2, W3T, b3):
    BK = 2048
    in_dim = NUM + R
    grid = (B // BK,)
    return pl.pallas_call(
        _mlp_body,
        grid=grid,
        in_specs=[
            pl.BlockSpec((NUM, BK), lambda i: (0, i)),
            pl.BlockSpec((R, BK), lambda i: (0, i)),
            pl.BlockSpec((64, in_dim), lambda i: (0, 0)),
            pl.BlockSpec((64, 1), lambda i: (0, 0)),
            pl.BlockSpec((32, 64), lambda i: (0, 0)),
            pl.BlockSpec((32, 1), lambda i: (0, 0)),
            pl.BlockSpec((1, 32), lambda i: (0, 0)),
            pl.BlockSpec((1, 1), lambda i: (0, 0)),
        ],
        out_specs=pl.BlockSpec((1, BK), lambda i: (0, i)),
        out_shape=jax.ShapeDtypeStruct((1, B), jnp.float32),
    )(xnT, embsT, W1T, b1.reshape(64, 1), W2T, b2.reshape(32, 1), W3T,
      b3.reshape(1, 1))


def kernel(x_num, x_cat, emb_tables, W1, b1, W2, b2, W3, b3):
    tblT = emb_tables.transpose(0, 2, 1).reshape(R, V)
    xcatT = x_cat.T.astype(jnp.int32)
    embsT = _sc_gather(tblT, xcatT)
    outT = _tc_mlp(x_num.T, embsT, W1.T, b1, W2.T, b2, W3.T, b3)
    return outT.reshape(B, 1)


# speedup vs baseline: 51.3410x; 1.3179x over previous
import functools

import jax
import jax.numpy as jnp
from jax import lax
from jax.experimental import pallas as pl
from jax.experimental.pallas import tpu as pltpu
from jax.experimental.pallas import tpu_sc as plsc

B = 16384
F = 26
V = 100000
D = 16
NUM = 16

NC = 2
NS = 16
NW = NC * NS
R = F * D
PW = R // NW
QC = 4096
NQ = B // QC
UNROLL = 8


def _gather_body(tbl_hbm, xcat_hbm, out_hbm, row_v, idx0, idx1, res0, res1,
                 rsem, is0, is1, os0, os1):
    wid = lax.axis_index("s") * NC + lax.axis_index("c")
    idx_b = (idx0, idx1)
    res_b = (res0, res1)
    isem = (is0, is1)
    osem = (os0, os1)

    def pair_body(j, _):
        p = wid * PW + j
        f = p >> 4
        row_cp = pltpu.async_copy(tbl_hbm.at[p], row_v, rsem)
        icp = [None, None]
        ocp = [None, None]
        icp[0] = pltpu.async_copy(xcat_hbm.at[f, pl.ds(0, QC)], idx0, is0)
        icp[1] = pltpu.async_copy(xcat_hbm.at[f, pl.ds(QC, QC)], idx1, is1)
        row_cp.wait()
        for t in range(NQ):
            b = t & 1
            icp[b].wait()
            if ocp[b] is not None:
                ocp[b].wait()
            iv = idx_b[b]
            rv = res_b[b]

            def gather_step(i, _):
                base = i * (16 * UNROLL)
                for k in range(UNROLL):
                    o = base + k * 16
                    vec = iv[pl.ds(o, 16)]
                    rv[pl.ds(o, 16)] = plsc.load_gather(row_v, [vec])
                return 0

            lax.fori_loop(0, QC // (16 * UNROLL), gather_step, 0)
            if t + 2 < NQ:
                icp[b] = pltpu.async_copy(
                    xcat_hbm.at[f, pl.ds((t + 2) * QC, QC)], iv, isem[b])
            ocp[b] = pltpu.async_copy(
                rv, out_hbm.at[p, pl.ds(t * QC, QC)], osem[b])
        ocp[0].wait()
        ocp[1].wait()
        return 0

    lax.fori_loop(0, PW, pair_body, 0)


def _sc_gather(tblT, xcatT):
    mesh = plsc.VectorSubcoreMesh(core_axis_name="c", subcore_axis_name="s")
    f = pl.kernel(
        _gather_body,
        out_type=jax.ShapeDtypeStruct((R, B), jnp.float32),
        mesh=mesh,
        compiler_params=pltpu.CompilerParams(use_tc_tiling_on_sc=True,
                                             needs_layout_passes=False),
        scratch_types=[
            pltpu.VMEM((V,), jnp.float32),
            pltpu.VMEM((QC,), jnp.int32),
            pltpu.VMEM((QC,), jnp.int32),
            pltpu.VMEM((QC,), jnp.float32),
            pltpu.VMEM((QC,), jnp.float32),
            pltpu.SemaphoreType.DMA,
            pltpu.SemaphoreType.DMA,
            pltpu.SemaphoreType.DMA,
            pltpu.SemaphoreType.DMA,
            pltpu.SemaphoreType.DMA,
        ],
    )
    return f(tblT, xcatT)


def _mlp_body(xnt_ref, et_ref, w1t_ref, b1_ref, w2t_ref, b2_ref, w3t_ref,
              b3_ref, o_ref):
    hp = jax.lax.Precision.HIGHEST
    h = jnp.dot(w1t_ref[:, :NUM], xnt_ref[...],
                preferred_element_type=jnp.float32, precision=hp)
    h = h + jnp.dot(w1t_ref[:, NUM:], et_ref[...],
                    preferred_element_type=jnp.float32, precision=hp)
    h = jnp.maximum(h + b1_ref[...], 0.0)
    h = jnp.dot(w2t_ref[...], h, preferred_element_type=jnp.float32,
                precision=hp)
    h = jnp.maximum(h + b2_ref[...], 0.0)
    o_ref[...] = jnp.dot(w3t_ref[...], h, preferred_element_type=jnp.float32,
                         precision=hp) + b3_ref[...]


def _tc_mlp(xnT, embsT, W1T, b1, W2T, b2, W3T, b3):
    BK = 2048
    in_dim = NUM + R
    grid = (B // BK,)
    return pl.pallas_call(
        _mlp_body,
        grid=grid,
        in_specs=[
            pl.BlockSpec((NUM, BK), lambda i: (0, i)),
            pl.BlockSpec((R, BK), lambda i: (0, i)),
            pl.BlockSpec((64, in_dim), lambda i: (0, 0)),
            pl.BlockSpec((64, 1), lambda i: (0, 0)),
            pl.BlockSpec((32, 64), lambda i: (0, 0)),
            pl.BlockSpec((32, 1), lambda i: (0, 0)),
            pl.BlockSpec((1, 32), lambda i: (0, 0)),
            pl.BlockSpec((1, 1), lambda i: (0, 0)),
        ],
        out_specs=pl.BlockSpec((1, BK), lambda i: (0, i)),
        out_shape=jax.ShapeDtypeStruct((1, B), jnp.float32),
    )(xnT, embsT, W1T, b1.reshape(64, 1), W2T, b2.reshape(32, 1), W3T,
      b3.reshape(1, 1))


def kernel(x_num, x_cat, emb_tables, W1, b1, W2, b2, W3, b3):
    tblT = emb_tables.transpose(0, 2, 1).reshape(R, V)
    xcatT = x_cat.T.astype(jnp.int32)
    embsT = _sc_gather(tblT, xcatT)
    outT = _tc_mlp(x_num.T, embsT, W1.T, b1, W2.T, b2, W3.T, b3)
    return outT.reshape(B, 1)


# speedup vs baseline: 51.4248x; 1.0016x over previous
import functools

import jax
import jax.numpy as jnp
from jax import lax
from jax.experimental import pallas as pl
from jax.experimental.pallas import tpu as pltpu
from jax.experimental.pallas import tpu_sc as plsc

B = 16384
F = 26
V = 100000
D = 16
NUM = 16

NC = 2
NS = 16
NW = NC * NS
R = F * D
PW = R // NW
QC = 4096
NQ = B // QC
UNROLL = 8


def _gather_body(tbl_hbm, xcat_hbm, out_hbm, row_v, idx0, idx1, res0, res1,
                 rsem, is0, is1, os0, os1):
    wid = lax.axis_index("s") * NC + lax.axis_index("c")
    idx_b = (idx0, idx1)
    res_b = (res0, res1)
    isem = (is0, is1)
    osem = (os0, os1)

    def pair_body(j, _):
        p = wid * PW + j
        f = p >> 4
        row_cp = pltpu.async_copy(tbl_hbm.at[p], row_v, rsem)
        icp = [None, None]
        ocp = [None, None]
        icp[0] = pltpu.async_copy(xcat_hbm.at[f, pl.ds(0, QC)], idx0, is0)
        icp[1] = pltpu.async_copy(xcat_hbm.at[f, pl.ds(QC, QC)], idx1, is1)
        row_cp.wait()
        for t in range(NQ):
            b = t & 1
            icp[b].wait()
            if ocp[b] is not None:
                ocp[b].wait()
            iv = idx_b[b]
            rv = res_b[b]

            def gather_step(i, _):
                base = i * (16 * UNROLL)
                for k in range(UNROLL):
                    o = base + k * 16
                    vec = iv[pl.ds(o, 16)]
                    rv[pl.ds(o, 16)] = plsc.load_gather(row_v, [vec])
                return 0

            lax.fori_loop(0, QC // (16 * UNROLL), gather_step, 0)
            if t + 2 < NQ:
                icp[b] = pltpu.async_copy(
                    xcat_hbm.at[f, pl.ds((t + 2) * QC, QC)], iv, isem[b])
            ocp[b] = pltpu.async_copy(
                rv, out_hbm.at[p, pl.ds(t * QC, QC)], osem[b])
        ocp[0].wait()
        ocp[1].wait()
        return 0

    lax.fori_loop(0, PW, pair_body, 0)


def _sc_gather(tblT, xcatT):
    mesh = plsc.VectorSubcoreMesh(core_axis_name="c", subcore_axis_name="s")
    f = pl.kernel(
        _gather_body,
        out_type=jax.ShapeDtypeStruct((R, B), jnp.float32),
        mesh=mesh,
        compiler_params=pltpu.CompilerParams(use_tc_tiling_on_sc=True,
                                             needs_layout_passes=False),
        scratch_types=[
            pltpu.VMEM((V,), jnp.float32),
            pltpu.VMEM((QC,), jnp.int32),
            pltpu.VMEM((QC,), jnp.int32),
            pltpu.VMEM((QC,), jnp.float32),
            pltpu.VMEM((QC,), jnp.float32),
            pltpu.SemaphoreType.DMA,
            pltpu.SemaphoreType.DMA,
            pltpu.SemaphoreType.DMA,
            pltpu.SemaphoreType.DMA,
            pltpu.SemaphoreType.DMA,
        ],
    )
    return f(tblT, xcatT)


def _mlp_body(xnt_ref, et_ref, w1t_ref, b1_ref, w2t_ref, b2_ref, w3t_ref,
              b3_ref, o_ref):
    hp = jax.lax.Precision.HIGHEST
    h = jnp.dot(w1t_ref[:, :NUM], xnt_ref[...],
                preferred_element_type=jnp.float32, precision=hp)
    h = h + jnp.dot(w1t_ref[:, NUM:], et_ref[...],
                    preferred_element_type=jnp.float32, precision=hp)
    h = jnp.maximum(h + b1_ref[...], 0.0)
    h = jnp.dot(w2t_ref[...], h, preferred_element_type=jnp.float32,
                precision=hp)
    h = jnp.maximum(h + b2_ref[...], 0.0)
    o_ref[...] = jnp.dot(w3t_ref[...], h, preferred_element_type=jnp.float32,
                         precision=hp) + b3_ref[...]


def _tc_mlp(xnT, embsT, W1T, b1, W2T, b2, W3T, b3):
    BK = 4096
    in_dim = NUM + R
    grid = (B // BK,)
    return pl.pallas_call(
        _mlp_body,
        grid=grid,
        in_specs=[
            pl.BlockSpec((NUM, BK), lambda i: (0, i)),
            pl.BlockSpec((R, BK), lambda i: (0, i)),
            pl.BlockSpec((64, in_dim), lambda i: (0, 0)),
            pl.BlockSpec((64, 1), lambda i: (0, 0)),
            pl.BlockSpec((32, 64), lambda i: (0, 0)),
            pl.BlockSpec((32, 1), lambda i: (0, 0)),
            pl.BlockSpec((1, 32), lambda i: (0, 0)),
            pl.BlockSpec((1, 1), lambda i: (0, 0)),
        ],
        out_specs=pl.BlockSpec((1, BK), lambda i: (0, i)),
        out_shape=jax.ShapeDtypeStruct((1, B), jnp.float32),
    )(xnT, embsT, W1T, b1.reshape(64, 1), W2T, b2.reshape(32, 1), W3T,
      b3.reshape(1, 1))


def kernel(x_num, x_cat, emb_tables, W1, b1, W2, b2, W3, b3):
    tblT = emb_tables.transpose(0, 2, 1).reshape(R, V)
    xcatT = x_cat.T.astype(jnp.int32)
    embsT = _sc_gather(tblT, xcatT)
    outT = _tc_mlp(x_num.T, embsT, W1.T, b1, W2.T, b2, W3.T, b3)
    return outT.reshape(B, 1)


# speedup vs baseline: 56.2651x; 1.0941x over previous
import functools

import jax
import jax.numpy as jnp
from jax import lax
from jax.experimental import pallas as pl
from jax.experimental.pallas import tpu as pltpu
from jax.experimental.pallas import tpu_sc as plsc

B = 16384
F = 26
V = 100000
D = 16
NUM = 16

NC = 2
NS = 16
NW = NC * NS
R = F * D
PW = R // NW
QC = 4096
NQ = B // QC
UNROLL = 8


def _gather_body(tbl_hbm, xcat_hbm, out_hbm, row_v, idx0, idx1, res0, res1,
                 rsem, is0, is1, os0, os1):
    wid = lax.axis_index("s") * NC + lax.axis_index("c")
    idx_b = (idx0, idx1)
    res_b = (res0, res1)
    isem = (is0, is1)
    osem = (os0, os1)

    def pair_body(j, _):
        p = wid * PW + j
        f = p >> 4
        row_cp = pltpu.async_copy(tbl_hbm.at[p], row_v, rsem)
        icp = [None, None]
        ocp = [None, None]
        icp[0] = pltpu.async_copy(xcat_hbm.at[f, pl.ds(0, QC)], idx0, is0)
        icp[1] = pltpu.async_copy(xcat_hbm.at[f, pl.ds(QC, QC)], idx1, is1)
        row_cp.wait()
        for t in range(NQ):
            b = t & 1
            icp[b].wait()
            if ocp[b] is not None:
                ocp[b].wait()
            iv = idx_b[b]
            rv = res_b[b]

            def gather_step(i, _):
                base = i * (16 * UNROLL)
                for k in range(UNROLL):
                    o = base + k * 16
                    vec = iv[pl.ds(o, 16)]
                    rv[pl.ds(o, 16)] = plsc.load_gather(row_v, [vec])
                return 0

            lax.fori_loop(0, QC // (16 * UNROLL), gather_step, 0)
            if t + 2 < NQ:
                icp[b] = pltpu.async_copy(
                    xcat_hbm.at[f, pl.ds((t + 2) * QC, QC)], iv, isem[b])
            ocp[b] = pltpu.async_copy(
                rv, out_hbm.at[p, pl.ds(t * QC, QC)], osem[b])
        ocp[0].wait()
        ocp[1].wait()
        return 0

    lax.fori_loop(0, PW, pair_body, 0)


def _sc_gather(tblT, xcatT):
    mesh = plsc.VectorSubcoreMesh(core_axis_name="c", subcore_axis_name="s")
    f = pl.kernel(
        _gather_body,
        out_type=jax.ShapeDtypeStruct((R, B), jnp.float32),
        mesh=mesh,
        compiler_params=pltpu.CompilerParams(use_tc_tiling_on_sc=True,
                                             needs_layout_passes=False),
        scratch_types=[
            pltpu.VMEM((V,), jnp.float32),
            pltpu.VMEM((QC,), jnp.int32),
            pltpu.VMEM((QC,), jnp.int32),
            pltpu.VMEM((QC,), jnp.float32),
            pltpu.VMEM((QC,), jnp.float32),
            pltpu.SemaphoreType.DMA,
            pltpu.SemaphoreType.DMA,
            pltpu.SemaphoreType.DMA,
            pltpu.SemaphoreType.DMA,
            pltpu.SemaphoreType.DMA,
        ],
    )
    return f(tblT, xcatT)


def _mlp_body(xnt_ref, et_ref, w1t_ref, b1_ref, w2t_ref, b2_ref, w3t_ref,
              b3_ref, o_ref):
    hp = jax.lax.Precision.DEFAULT
    h = jnp.dot(w1t_ref[:, :NUM], xnt_ref[...],
                preferred_element_type=jnp.float32, precision=hp)
    h = h + jnp.dot(w1t_ref[:, NUM:], et_ref[...],
                    preferred_element_type=jnp.float32, precision=hp)
    h = jnp.maximum(h + b1_ref[...], 0.0)
    h = jnp.dot(w2t_ref[...], h, preferred_element_type=jnp.float32,
                precision=hp)
    h = jnp.maximum(h + b2_ref[...], 0.0)
    o_ref[...] = jnp.dot(w3t_ref[...], h, preferred_element_type=jnp.float32,
                         precision=hp) + b3_ref[...]


def _tc_mlp(xnT, embsT, W1T, b1, W2T, b2, W3T, b3):
    BK = 4096
    in_dim = NUM + R
    grid = (B // BK,)
    return pl.pallas_call(
        _mlp_body,
        grid=grid,
        in_specs=[
            pl.BlockSpec((NUM, BK), lambda i: (0, i)),
            pl.BlockSpec((R, BK), lambda i: (0, i)),
            pl.BlockSpec((64, in_dim), lambda i: (0, 0)),
            pl.BlockSpec((64, 1), lambda i: (0, 0)),
            pl.BlockSpec((32, 64), lambda i: (0, 0)),
            pl.BlockSpec((32, 1), lambda i: (0, 0)),
            pl.BlockSpec((1, 32), lambda i: (0, 0)),
            pl.BlockSpec((1, 1), lambda i: (0, 0)),
        ],
        out_specs=pl.BlockSpec((1, BK), lambda i: (0, i)),
        out_shape=jax.ShapeDtypeStruct((1, B), jnp.float32),
    )(xnT, embsT, W1T, b1.reshape(64, 1), W2T, b2.reshape(32, 1), W3T,
      b3.reshape(1, 1))


def kernel(x_num, x_cat, emb_tables, W1, b1, W2, b2, W3, b3):
    tblT = emb_tables.transpose(0, 2, 1).reshape(R, V)
    xcatT = x_cat.T.astype(jnp.int32)
    embsT = _sc_gather(tblT, xcatT)
    outT = _tc_mlp(x_num.T, embsT, W1.T, b1, W2.T, b2, W3.T, b3)
    return outT.reshape(B, 1)


# speedup vs baseline: 56.3110x; 1.0008x over previous
import functools

import jax
import jax.numpy as jnp
from jax import lax
from jax.experimental import pallas as pl
from jax.experimental.pallas import tpu as pltpu
from jax.experimental.pallas import tpu_sc as plsc

B = 16384
F = 26
V = 100000
D = 16
NUM = 16

NC = 2
NS = 16
NW = NC * NS
R = F * D
PW = R // NW
QC = 4096
NQ = B // QC
UNROLL = 16


def _gather_body(tbl_hbm, xcat_hbm, out_hbm, row_v, idx0, idx1, res0, res1,
                 rsem, is0, is1, os0, os1):
    wid = lax.axis_index("s") * NC + lax.axis_index("c")
    idx_b = (idx0, idx1)
    res_b = (res0, res1)
    isem = (is0, is1)
    osem = (os0, os1)

    def pair_body(j, _):
        p = wid * PW + j
        f = p >> 4
        row_cp = pltpu.async_copy(tbl_hbm.at[p], row_v, rsem)
        icp = [None, None]
        ocp = [None, None]
        icp[0] = pltpu.async_copy(xcat_hbm.at[f, pl.ds(0, QC)], idx0, is0)
        icp[1] = pltpu.async_copy(xcat_hbm.at[f, pl.ds(QC, QC)], idx1, is1)
        row_cp.wait()
        for t in range(NQ):
            b = t & 1
            icp[b].wait()
            if ocp[b] is not None:
                ocp[b].wait()
            iv = idx_b[b]
            rv = res_b[b]

            def gather_step(i, _):
                base = i * (16 * UNROLL)
                for k in range(UNROLL):
                    o = base + k * 16
                    vec = iv[pl.ds(o, 16)]
                    rv[pl.ds(o, 16)] = plsc.load_gather(row_v, [vec])
                return 0

            lax.fori_loop(0, QC // (16 * UNROLL), gather_step, 0)
            if t + 2 < NQ:
                icp[b] = pltpu.async_copy(
                    xcat_hbm.at[f, pl.ds((t + 2) * QC, QC)], iv, isem[b])
            ocp[b] = pltpu.async_copy(
                rv, out_hbm.at[p, pl.ds(t * QC, QC)], osem[b])
        ocp[0].wait()
        ocp[1].wait()
        return 0

    lax.fori_loop(0, PW, pair_body, 0)


def _sc_gather(tblT, xcatT):
    mesh = plsc.VectorSubcoreMesh(core_axis_name="c", subcore_axis_name="s")
    f = pl.kernel(
        _gather_body,
        out_type=jax.ShapeDtypeStruct((R, B), jnp.float32),
        mesh=mesh,
        compiler_params=pltpu.CompilerParams(use_tc_tiling_on_sc=True,
                                             needs_layout_passes=False),
        scratch_types=[
            pltpu.VMEM((V,), jnp.float32),
            pltpu.VMEM((QC,), jnp.int32),
            pltpu.VMEM((QC,), jnp.int32),
            pltpu.VMEM((QC,), jnp.float32),
            pltpu.VMEM((QC,), jnp.float32),
            pltpu.SemaphoreType.DMA,
            pltpu.SemaphoreType.DMA,
            pltpu.SemaphoreType.DMA,
            pltpu.SemaphoreType.DMA,
            pltpu.SemaphoreType.DMA,
        ],
    )
    return f(tblT, xcatT)


def _mlp_body(xnt_ref, et_ref, w1t_ref, b1_ref, w2t_ref, b2_ref, w3t_ref,
              b3_ref, o_ref):
    hp = jax.lax.Precision.DEFAULT
    h = jnp.dot(w1t_ref[:, :NUM], xnt_ref[...],
                preferred_element_type=jnp.float32, precision=hp)
    h = h + jnp.dot(w1t_ref[:, NUM:], et_ref[...],
                    preferred_element_type=jnp.float32, precision=hp)
    h = jnp.maximum(h + b1_ref[...], 0.0)
    h = jnp.dot(w2t_ref[...], h, preferred_element_type=jnp.float32,
                precision=hp)
    h = jnp.maximum(h + b2_ref[...], 0.0)
    o_ref[...] = jnp.dot(w3t_ref[...], h, preferred_element_type=jnp.float32,
                         precision=hp) + b3_ref[...]


def _tc_mlp(xnT, embsT, W1T, b1, W2T, b2, W3T, b3):
    BK = 4096
    in_dim = NUM + R
    grid = (B // BK,)
    return pl.pallas_call(
        _mlp_body,
        grid=grid,
        in_specs=[
            pl.BlockSpec((NUM, BK), lambda i: (0, i)),
            pl.BlockSpec((R, BK), lambda i: (0, i)),
            pl.BlockSpec((64, in_dim), lambda i: (0, 0)),
            pl.BlockSpec((64, 1), lambda i: (0, 0)),
            pl.BlockSpec((32, 64), lambda i: (0, 0)),
            pl.BlockSpec((32, 1), lambda i: (0, 0)),
            pl.BlockSpec((1, 32), lambda i: (0, 0)),
            pl.BlockSpec((1, 1), lambda i: (0, 0)),
        ],
        out_specs=pl.BlockSpec((1, BK), lambda i: (0, i)),
        out_shape=jax.ShapeDtypeStruct((1, B), jnp.float32),
    )(xnT, embsT, W1T, b1.reshape(64, 1), W2T, b2.reshape(32, 1), W3T,
      b3.reshape(1, 1))


def kernel(x_num, x_cat, emb_tables, W1, b1, W2, b2, W3, b3):
    tblT = emb_tables.transpose(0, 2, 1).reshape(R, V)
    xcatT = x_cat.T.astype(jnp.int32)
    embsT = _sc_gather(tblT, xcatT)
    outT = _tc_mlp(x_num.T, embsT, W1.T, b1, W2.T, b2, W3.T, b3)
    return outT.reshape(B, 1)


# speedup vs baseline: 56.5998x; 1.0051x over previous
import functools

import jax
import jax.numpy as jnp
from jax import lax
from jax.experimental import pallas as pl
from jax.experimental.pallas import tpu as pltpu
from jax.experimental.pallas import tpu_sc as plsc

B = 16384
F = 26
V = 100000
D = 16
NUM = 16

NC = 2
NS = 16
NW = NC * NS
R = F * D
PW = R // NW
QC = 4096
NQ = B // QC
UNROLL = 16


def _gather_body(tbl_hbm, xcat_hbm, out_hbm, row_v, idx0, idx1, res0, res1,
                 rsem, is0, is1, os0, os1):
    wid = lax.axis_index("s") * NC + lax.axis_index("c")
    @pl.when((lax.axis_index("s") & 1) == 1)
    def _stagger():
        pl.delay(3400)

    idx_b = (idx0, idx1)
    res_b = (res0, res1)
    isem = (is0, is1)
    osem = (os0, os1)

    def pair_body(j, _):
        p = wid * PW + j
        f = p >> 4
        row_cp = pltpu.async_copy(tbl_hbm.at[p], row_v, rsem)
        icp = [None, None]
        ocp = [None, None]
        icp[0] = pltpu.async_copy(xcat_hbm.at[f, pl.ds(0, QC)], idx0, is0)
        icp[1] = pltpu.async_copy(xcat_hbm.at[f, pl.ds(QC, QC)], idx1, is1)
        row_cp.wait()
        for t in range(NQ):
            b = t & 1
            icp[b].wait()
            if ocp[b] is not None:
                ocp[b].wait()
            iv = idx_b[b]
            rv = res_b[b]

            def gather_step(i, _):
                base = i * (16 * UNROLL)
                for k in range(UNROLL):
                    o = base + k * 16
                    vec = iv[pl.ds(o, 16)]
                    rv[pl.ds(o, 16)] = plsc.load_gather(row_v, [vec])
                return 0

            lax.fori_loop(0, QC // (16 * UNROLL), gather_step, 0)
            if t + 2 < NQ:
                icp[b] = pltpu.async_copy(
                    xcat_hbm.at[f, pl.ds((t + 2) * QC, QC)], iv, isem[b])
            ocp[b] = pltpu.async_copy(
                rv, out_hbm.at[p, pl.ds(t * QC, QC)], osem[b])
        ocp[0].wait()
        ocp[1].wait()
        return 0

    lax.fori_loop(0, PW, pair_body, 0)


def _sc_gather(tblT, xcatT):
    mesh = plsc.VectorSubcoreMesh(core_axis_name="c", subcore_axis_name="s")
    f = pl.kernel(
        _gather_body,
        out_type=jax.ShapeDtypeStruct((R, B), jnp.float32),
        mesh=mesh,
        compiler_params=pltpu.CompilerParams(use_tc_tiling_on_sc=True,
                                             needs_layout_passes=False),
        scratch_types=[
            pltpu.VMEM((V,), jnp.float32),
            pltpu.VMEM((QC,), jnp.int32),
            pltpu.VMEM((QC,), jnp.int32),
            pltpu.VMEM((QC,), jnp.float32),
            pltpu.VMEM((QC,), jnp.float32),
            pltpu.SemaphoreType.DMA,
            pltpu.SemaphoreType.DMA,
            pltpu.SemaphoreType.DMA,
            pltpu.SemaphoreType.DMA,
            pltpu.SemaphoreType.DMA,
        ],
    )
    return f(tblT, xcatT)


def _mlp_body(xnt_ref, et_ref, w1t_ref, b1_ref, w2t_ref, b2_ref, w3t_ref,
              b3_ref, o_ref):
    hp = jax.lax.Precision.DEFAULT
    h = jnp.dot(w1t_ref[:, :NUM], xnt_ref[...],
                preferred_element_type=jnp.float32, precision=hp)
    h = h + jnp.dot(w1t_ref[:, NUM:], et_ref[...],
                    preferred_element_type=jnp.float32, precision=hp)
    h = jnp.maximum(h + b1_ref[...], 0.0)
    h = jnp.dot(w2t_ref[...], h, preferred_element_type=jnp.float32,
                precision=hp)
    h = jnp.maximum(h + b2_ref[...], 0.0)
    o_ref[...] = jnp.dot(w3t_ref[...], h, preferred_element_type=jnp.float32,
                         precision=hp) + b3_ref[...]


def _tc_mlp(xnT, embsT, W1T, b1, W2T, b2, W3T, b3):
    BK = 4096
    in_dim = NUM + R
    grid = (B // BK,)
    return pl.pallas_call(
        _mlp_body,
        grid=grid,
        in_specs=[
            pl.BlockSpec((NUM, BK), lambda i: (0, i)),
            pl.BlockSpec((R, BK), lambda i: (0, i)),
            pl.BlockSpec((64, in_dim), lambda i: (0, 0)),
            pl.BlockSpec((64, 1), lambda i: (0, 0)),
            pl.BlockSpec((32, 64), lambda i: (0, 0)),
            pl.BlockSpec((32, 1), lambda i: (0, 0)),
            pl.BlockSpec((1, 32), lambda i: (0, 0)),
            pl.BlockSpec((1, 1), lambda i: (0, 0)),
        ],
        out_specs=pl.BlockSpec((1, BK), lambda i: (0, i)),
        out_shape=jax.ShapeDtypeStruct((1, B), jnp.float32),
    )(xnT, embsT, W1T, b1.reshape(64, 1), W2T, b2.reshape(32, 1), W3T,
      b3.reshape(1, 1))


def kernel(x_num, x_cat, emb_tables, W1, b1, W2, b2, W3, b3):
    tblT = emb_tables.transpose(0, 2, 1).reshape(R, V)
    xcatT = x_cat.T.astype(jnp.int32)
    embsT = _sc_gather(tblT, xcatT)
    outT = _tc_mlp(x_num.T, embsT, W1.T, b1, W2.T, b2, W3.T, b3)
    return outT.reshape(B, 1)


# speedup vs baseline: 57.4406x; 1.0149x over previous
import functools

import jax
import jax.numpy as jnp
from jax import lax
from jax.experimental import pallas as pl
from jax.experimental.pallas import tpu as pltpu
from jax.experimental.pallas import tpu_sc as plsc

B = 16384
F = 26
V = 100000
D = 16
NUM = 16

NC = 2
NS = 16
NW = NC * NS
R = F * D
PW = R // NW
QC = 4096
NQ = B // QC
UNROLL = 16


def _gather_body(tbl_hbm, xcat_hbm, out_hbm, row_v, idx_v, res0, res1,
                 rsem, isem, os0, os1):
    wid = lax.axis_index("s") * NC + lax.axis_index("c")
    @pl.when((lax.axis_index("s") & 1) == 1)
    def _stagger():
        pl.delay(3400)

    res_b = (res0, res1)
    osem = (os0, os1)

    def pair_body(j, _):
        p = wid * PW + j
        f = p >> 4
        row_cp = pltpu.async_copy(tbl_hbm.at[p], row_v, rsem)

        @pl.when((j == 0) | ((p & 15) == 0))
        def _load_idx():
            pltpu.async_copy(xcat_hbm.at[f], idx_v, isem).wait()

        row_cp.wait()
        ocp = [None, None]
        for t in range(NQ):
            b = t & 1
            if ocp[b] is not None:
                ocp[b].wait()
            rv = res_b[b]

            def gather_step(i, _):
                base = t * QC + i * (16 * UNROLL)
                for k in range(UNROLL):
                    o = base + k * 16
                    vec = idx_v[pl.ds(o, 16)]
                    rv[pl.ds(o - t * QC, 16)] = plsc.load_gather(row_v, [vec])
                return 0

            lax.fori_loop(0, QC // (16 * UNROLL), gather_step, 0)
            ocp[b] = pltpu.async_copy(
                rv, out_hbm.at[p, pl.ds(t * QC, QC)], osem[b])
        ocp[0].wait()
        ocp[1].wait()
        return 0

    lax.fori_loop(0, PW, pair_body, 0)


def _sc_gather(tblT, xcatT):
    mesh = plsc.VectorSubcoreMesh(core_axis_name="c", subcore_axis_name="s")
    f = pl.kernel(
        _gather_body,
        out_type=jax.ShapeDtypeStruct((R, B), jnp.float32),
        mesh=mesh,
        compiler_params=pltpu.CompilerParams(use_tc_tiling_on_sc=True,
                                             needs_layout_passes=False),
        scratch_types=[
            pltpu.VMEM((V,), jnp.float32),
            pltpu.VMEM((B,), jnp.int32),
            pltpu.VMEM((QC,), jnp.float32),
            pltpu.VMEM((QC,), jnp.float32),
            pltpu.SemaphoreType.DMA,
            pltpu.SemaphoreType.DMA,
            pltpu.SemaphoreType.DMA,
            pltpu.SemaphoreType.DMA,
        ],
    )
    return f(tblT, xcatT)


def _mlp_body(xnt_ref, et_ref, w1t_ref, b1_ref, w2t_ref, b2_ref, w3t_ref,
              b3_ref, o_ref):
    hp = jax.lax.Precision.DEFAULT
    h = jnp.dot(w1t_ref[:, :NUM], xnt_ref[...],
                preferred_element_type=jnp.float32, precision=hp)
    h = h + jnp.dot(w1t_ref[:, NUM:], et_ref[...],
                    preferred_element_type=jnp.float32, precision=hp)
    h = jnp.maximum(h + b1_ref[...], 0.0)
    h = jnp.dot(w2t_ref[...], h, preferred_element_type=jnp.float32,
                precision=hp)
    h = jnp.maximum(h + b2_ref[...], 0.0)
    o_ref[...] = jnp.dot(w3t_ref[...], h, preferred_element_type=jnp.float32,
                         precision=hp) + b3_ref[...]


def _tc_mlp(xnT, embsT, W1T, b1, W2T, b2, W3T, b3):
    BK = 4096
    in_dim = NUM + R
    grid = (B // BK,)
    return pl.pallas_call(
        _mlp_body,
        grid=grid,
        in_specs=[
            pl.BlockSpec((NUM, BK), lambda i: (0, i)),
            pl.BlockSpec((R, BK), lambda i: (0, i)),
            pl.BlockSpec((64, in_dim), lambda i: (0, 0)),
            pl.BlockSpec((64, 1), lambda i: (0, 0)),
            pl.BlockSpec((32, 64), lambda i: (0, 0)),
            pl.BlockSpec((32, 1), lambda i: (0, 0)),
            pl.BlockSpec((1, 32), lambda i: (0, 0)),
            pl.BlockSpec((1, 1), lambda i: (0, 0)),
        ],
        out_specs=pl.BlockSpec((1, BK), lambda i: (0, i)),
        out_shape=jax.ShapeDtypeStruct((1, B), jnp.float32),
    )(xnT, embsT, W1T, b1.reshape(64, 1), W2T, b2.reshape(32, 1), W3T,
      b3.reshape(1, 1))


def kernel(x_num, x_cat, emb_tables, W1, b1, W2, b2, W3, b3):
    tblT = emb_tables.transpose(0, 2, 1).reshape(R, V)
    xcatT = x_cat.T.astype(jnp.int32)
    embsT = _sc_gather(tblT, xcatT)
    outT = _tc_mlp(x_num.T, embsT, W1.T, b1, W2.T, b2, W3.T, b3)
    return outT.reshape(B, 1)
